# baseline (device time: 555864 ns/iter reference)
import jax
import jax.numpy as jnp
from jax import lax
from jax.experimental import pallas as pl
from jax.experimental.pallas import tpu as pltpu

P = 4
NC = 4


def kernel(x):
    m, n = x.shape
    blk = n // P
    ch = m // NC

    def body(
        x_ref, o_ref, rb_ref,
        vsend, vin_a, vin_b, vout_b,
        local_sem, a_in_sems, b_in_sem, b_out_sems,
        send_sems, recv_sems,
    ):
        my_x = lax.axis_index("x")
        my_y = lax.axis_index("y")
        my_z = lax.axis_index("z")

        barrier = pltpu.get_barrier_semaphore()
        for dy in range(1, P):
            peer = (my_y + dy) % P
            pl.semaphore_signal(
                barrier, inc=1,
                device_id=(my_x, peer, my_z),
                device_id_type=pl.DeviceIdType.MESH,
            )

        local = pltpu.make_async_copy(
            x_ref.at[:, pl.ds(my_y * blk, blk)],
            o_ref.at[pl.ds(my_y * m, m), :],
            local_sem,
        )
        local.start()

        loads = {}

        def start_load(r, c):
            peer = my_y ^ r
            s = (r * NC + c) % 2
            cp = pltpu.make_async_copy(
                x_ref.at[pl.ds(c * ch, ch), pl.ds(peer * blk, blk)],
                vin_a.at[s],
                a_in_sems.at[s],
            )
            cp.start()
            loads[(r, c)] = cp

        def convert_block(r):
            for c in range(NC):
                loads[(r, c)].wait()
                vsend[r - 1, pl.ds(c * ch, ch), :] = vin_a[
                    (r * NC + c) % 2
                ].astype(jnp.bfloat16)
                nxt = r * NC + c + 2
                if nxt < 4 * NC:
                    start_load(nxt // NC, nxt % NC)

        start_load(1, 0)
        start_load(1, 1)
        convert_block(1)
        pl.semaphore_wait(barrier, P - 1)

        stores = {}
        js = 0

        for r in range(1, P):
            peer = my_y ^ r
            rdmas = []
            for c in range(NC):
                rdma = pltpu.make_async_remote_copy(
                    src_ref=vsend.at[r - 1, pl.ds(c * ch, ch), :],
                    dst_ref=rb_ref.at[r - 1, pl.ds(c * ch, ch), :],
                    send_sem=send_sems.at[r - 1, c],
                    recv_sem=recv_sems.at[r - 1, c],
                    device_id=(my_x, peer, my_z),
                    device_id_type=pl.DeviceIdType.MESH,
                )
                rdma.start()
                rdmas.append(rdma)

            if r + 1 < P:
                convert_block(r + 1)

            for c in range(NC):
                rdmas[c].wait_recv()
                load = pltpu.make_async_copy(
                    rb_ref.at[r - 1, pl.ds(c * ch, ch), :], vin_b, b_in_sem
                )
                load.start()
                load.wait()
                if js >= 2:
                    stores[js - 2].wait()
                vout_b[js % 2, :, :] = vin_b[...].astype(jnp.float32)
                st = pltpu.make_async_copy(
                    vout_b.at[js % 2],
                    o_ref.at[pl.ds(peer * m + c * ch, ch), :],
                    b_out_sems.at[js % 2],
                )
                st.start()
                stores[js] = st
                js += 1
            for c in range(NC):
                rdmas[c].wait_send()

        stores[js - 2].wait()
        stores[js - 1].wait()
        local.wait()

    out, _ = pl.pallas_call(
        body,
        out_shape=(
            jax.ShapeDtypeStruct((P * m, blk), x.dtype),
            jax.ShapeDtypeStruct((P - 1, m, blk), jnp.bfloat16),
        ),
        in_specs=[pl.BlockSpec(memory_space=pl.ANY)],
        out_specs=(
            pl.BlockSpec(memory_space=pl.ANY),
            pl.BlockSpec(memory_space=pl.ANY),
        ),
        scratch_shapes=[
            pltpu.MemorySpace.VMEM((P - 1, m, blk), jnp.bfloat16),
            pltpu.MemorySpace.VMEM((2, ch, blk), jnp.float32),
            pltpu.MemorySpace.VMEM((ch, blk), jnp.bfloat16),
            pltpu.MemorySpace.VMEM((2, ch, blk), jnp.float32),
            pltpu.SemaphoreType.DMA,
            pltpu.SemaphoreType.DMA((2,)),
            pltpu.SemaphoreType.DMA,
            pltpu.SemaphoreType.DMA((2,)),
            pltpu.SemaphoreType.DMA((P - 1, NC)),
            pltpu.SemaphoreType.DMA((P - 1, NC)),
        ],
        compiler_params=pltpu.CompilerParams(
            collective_id=0, vmem_limit_bytes=56 * 1024 * 1024
        ),
    )(x)
    return out


# device time: 554822 ns/iter; 1.0019x vs baseline; 1.0019x over previous
import jax
import jax.numpy as jnp
from jax import lax
from jax.experimental import pallas as pl
from jax.experimental.pallas import tpu as pltpu

P = 4
NC = 4


def kernel(x):
    m, n = x.shape
    blk = n // P
    ch = m // NC
    ch2 = ch // 2

    seq = [(dy, c) for c in range(NC) for dy in range(1, P)]

    def body(
        x_ref, o_ref, rb_ref,
        vsend, vin_a, vin_b, vout_b,
        local_sem, a_in_sems, b_in_sem, b_out_sems,
        send_sems, recv_sems,
    ):
        my_x = lax.axis_index("x")
        my_y = lax.axis_index("y")
        my_z = lax.axis_index("z")

        barrier = pltpu.get_barrier_semaphore()
        for dy in range(1, P):
            peer = (my_y + dy) % P
            pl.semaphore_signal(
                barrier, inc=1,
                device_id=(my_x, peer, my_z),
                device_id_type=pl.DeviceIdType.MESH,
            )

        local = pltpu.make_async_copy(
            x_ref.at[:, pl.ds(my_y * blk, blk)],
            o_ref.at[pl.ds(my_y * m, m), :],
            local_sem,
        )
        local.start()

        loads = {}

        def start_load(i):
            dy, c = seq[i]
            peer = (my_y + dy) % P
            cp = pltpu.make_async_copy(
                x_ref.at[pl.ds(c * ch, ch), pl.ds(peer * blk, blk)],
                vin_a.at[i % 2],
                a_in_sems.at[i % 2],
            )
            cp.start()
            loads[i] = cp

        start_load(0)
        start_load(1)
        pl.semaphore_wait(barrier, P - 1)

        rdmas = {}
        for i, (dy, c) in enumerate(seq):
            peer = (my_y + dy) % P
            loads[i].wait()
            vsend[dy - 1, pl.ds(c * ch2, ch2), :] = pltpu.bitcast(
                vin_a[i % 2].astype(jnp.bfloat16), jnp.float32
            )
            if i + 2 < len(seq):
                start_load(i + 2)
            rdma = pltpu.make_async_remote_copy(
                src_ref=vsend.at[dy - 1, pl.ds(c * ch2, ch2), :],
                dst_ref=rb_ref.at[dy - 1, pl.ds(c * ch2, ch2), :],
                send_sem=send_sems.at[dy - 1, c],
                recv_sem=recv_sems.at[dy - 1, c],
                device_id=(my_x, peer, my_z),
                device_id_type=pl.DeviceIdType.MESH,
            )
            rdma.start()
            rdmas[(dy, c)] = rdma

        stores = {}
        for j, (dy, c) in enumerate(seq):
            src_y = (my_y - dy) % P
            rdmas[(dy, c)].wait_recv()
            load = pltpu.make_async_copy(
                rb_ref.at[dy - 1, pl.ds(c * ch2, ch2), :], vin_b, b_in_sem
            )
            load.start()
            load.wait()
            if j >= 2:
                stores[j - 2].wait()
            vout_b[j % 2, :, :] = pltpu.bitcast(
                vin_b[...], jnp.bfloat16
            ).astype(jnp.float32)
            st = pltpu.make_async_copy(
                vout_b.at[j % 2],
                o_ref.at[pl.ds(src_y * m + c * ch, ch), :],
                b_out_sems.at[j % 2],
            )
            st.start()
            stores[j] = st

        stores[len(seq) - 2].wait()
        stores[len(seq) - 1].wait()
        local.wait()
        for rdma in rdmas.values():
            rdma.wait_send()

    out, _ = pl.pallas_call(
        body,
        out_shape=(
            jax.ShapeDtypeStruct((P * m, blk), x.dtype),
            jax.ShapeDtypeStruct((P - 1, m // 2, blk), jnp.float32),
        ),
        in_specs=[pl.BlockSpec(memory_space=pl.ANY)],
        out_specs=(
            pl.BlockSpec(memory_space=pl.ANY),
            pl.BlockSpec(memory_space=pl.ANY),
        ),
        scratch_shapes=[
            pltpu.MemorySpace.VMEM((P - 1, m // 2, blk), jnp.float32),
            pltpu.MemorySpace.VMEM((2, ch, blk), jnp.float32),
            pltpu.MemorySpace.VMEM((ch2, blk), jnp.float32),
            pltpu.MemorySpace.VMEM((2, ch, blk), jnp.float32),
            pltpu.SemaphoreType.DMA,
            pltpu.SemaphoreType.DMA((2,)),
            pltpu.SemaphoreType.DMA,
            pltpu.SemaphoreType.DMA((2,)),
            pltpu.SemaphoreType.DMA((P - 1, NC)),
            pltpu.SemaphoreType.DMA((P - 1, NC)),
        ],
        compiler_params=pltpu.CompilerParams(
            collective_id=0, vmem_limit_bytes=56 * 1024 * 1024
        ),
    )(x)
    return out


# device time: 416142 ns/iter; 1.3358x vs baseline; 1.3333x over previous
import jax
import jax.numpy as jnp
from jax import lax
from jax.experimental import pallas as pl
from jax.experimental.pallas import tpu as pltpu

P = 4
NC = 8


def kernel(x):
    m, n = x.shape
    blk = n // P
    ch = m // NC

    def body(
        x_ref, o_ref, rb_ref,
        vin, vsend, vloc, vcol, vin_b, vout_b,
        in_sems, loc_sems, col_sems, b_in_sem, b_out_sems,
        send_sems, recv_sems,
    ):
        my_x = lax.axis_index("x")
        my_y = lax.axis_index("y")
        my_z = lax.axis_index("z")

        barrier = pltpu.get_barrier_semaphore()
        for dy in range(1, P):
            peer = (my_y + dy) % P
            pl.semaphore_signal(
                barrier, inc=1,
                device_id=(my_x, peer, my_z),
                device_id_type=pl.DeviceIdType.MESH,
            )

        loads = {}

        def start_load(c):
            cp = pltpu.make_async_copy(
                x_ref.at[pl.ds(c * ch, ch), :],
                vin.at[c % 2],
                in_sems.at[c % 2],
            )
            cp.start()
            loads[c] = cp

        start_load(0)
        start_load(1)
        pl.semaphore_wait(barrier, P - 1)

        rdmas = {}
        loc_stores = {}
        for c in range(NC):
            loads[c].wait()
            if c >= 2:
                loc_stores[c - 2].wait()
            colcps = []
            loccp = pltpu.make_async_copy(
                vin.at[c % 2, :, pl.ds(my_y * blk, blk)],
                vloc.at[c % 2],
                loc_sems.at[c % 2],
            )
            loccp.start()
            for dy in range(1, P):
                peer = (my_y + dy) % P
                cp = pltpu.make_async_copy(
                    vin.at[c % 2, :, pl.ds(peer * blk, blk)],
                    vcol.at[dy - 1],
                    col_sems.at[dy - 1],
                )
                cp.start()
                colcps.append(cp)
            loccp.wait()
            st = pltpu.make_async_copy(
                vloc.at[c % 2],
                o_ref.at[pl.ds(my_y * m + c * ch, ch), :],
                loc_sems.at[c % 2],
            )
            st.start()
            loc_stores[c] = st
            for dy in range(1, P):
                colcps[dy - 1].wait()
                vsend[dy - 1, pl.ds(c * ch, ch), :] = vcol[dy - 1].astype(
                    jnp.bfloat16
                )
                peer = (my_y + dy) % P
                rdma = pltpu.make_async_remote_copy(
                    src_ref=vsend.at[dy - 1, pl.ds(c * ch, ch), :],
                    dst_ref=rb_ref.at[dy - 1, pl.ds(c * ch, ch), :],
                    send_sem=send_sems.at[dy - 1, c],
                    recv_sem=recv_sems.at[dy - 1, c],
                    device_id=(my_x, peer, my_z),
                    device_id_type=pl.DeviceIdType.MESH,
                )
                rdma.start()
                rdmas[(dy, c)] = rdma
            if c + 2 < NC:
                start_load(c + 2)

        stores = {}
        js = 0
        for c in range(NC):
            for dy in range(1, P):
                src_y = (my_y - dy) % P
                rdmas[(dy, c)].wait_recv()
                load = pltpu.make_async_copy(
                    rb_ref.at[dy - 1, pl.ds(c * ch, ch), :], vin_b, b_in_sem
                )
                load.start()
                load.wait()
                if js >= 2:
                    stores[js - 2].wait()
                vout_b[js % 2, :, :] = vin_b[...].astype(jnp.float32)
                st = pltpu.make_async_copy(
                    vout_b.at[js % 2],
                    o_ref.at[pl.ds(src_y * m + c * ch, ch), :],
                    b_out_sems.at[js % 2],
                )
                st.start()
                stores[js] = st
                js += 1

        stores[js - 2].wait()
        stores[js - 1].wait()
        loc_stores[NC - 2].wait()
        loc_stores[NC - 1].wait()
        for rdma in rdmas.values():
            rdma.wait_send()

    out, _ = pl.pallas_call(
        body,
        out_shape=(
            jax.ShapeDtypeStruct((P * m, blk), x.dtype),
            jax.ShapeDtypeStruct((P - 1, m, blk), jnp.bfloat16),
        ),
        in_specs=[pl.BlockSpec(memory_space=pl.ANY)],
        out_specs=(
            pl.BlockSpec(memory_space=pl.ANY),
            pl.BlockSpec(memory_space=pl.ANY),
        ),
        scratch_shapes=[
            pltpu.MemorySpace.VMEM((2, ch, n), jnp.float32),
            pltpu.MemorySpace.VMEM((P - 1, m, blk), jnp.bfloat16),
            pltpu.MemorySpace.VMEM((2, ch, blk), jnp.float32),
            pltpu.MemorySpace.VMEM((P - 1, ch, blk), jnp.float32),
            pltpu.MemorySpace.VMEM((ch, blk), jnp.bfloat16),
            pltpu.MemorySpace.VMEM((2, ch, blk), jnp.float32),
            pltpu.SemaphoreType.DMA((2,)),
            pltpu.SemaphoreType.DMA((2,)),
            pltpu.SemaphoreType.DMA((P - 1,)),
            pltpu.SemaphoreType.DMA,
            pltpu.SemaphoreType.DMA((2,)),
            pltpu.SemaphoreType.DMA((P - 1, NC)),
            pltpu.SemaphoreType.DMA((P - 1, NC)),
        ],
        compiler_params=pltpu.CompilerParams(
            collective_id=0, vmem_limit_bytes=60 * 1024 * 1024
        ),
    )(x)
    return out
